# baseline (speedup 1.0000x reference)
import jax
import jax.numpy as jnp
from jax.experimental import pallas as pl
from jax.experimental.pallas import tpu as pltpu

_N_LAT, _N_LON, _N_MESH, _N_NEI = 64, 128, 128, 4
_IN_DIM, _HID, _OUT_CH = 256, 256, 78
_BATCH = 32
_N_GRID = _N_LAT * _N_LON
_GLAT = 8
_GB = _GLAT * _N_LON
_MBB = 8


def _mlp_body(nf_ref, w1_ref, b1_ref, w2_ref, b2_ref, out_ref):
    x = nf_ref[...].reshape(_MBB * _N_MESH, _IN_DIM)
    h = jnp.dot(x, w1_ref[...],
                preferred_element_type=jnp.float32,
                precision=jax.lax.Precision.HIGHEST)
    h = jnp.maximum(h + b1_ref[...], 0.0)
    o = jnp.dot(h, w2_ref[...],
                preferred_element_type=jnp.float32,
                precision=jax.lax.Precision.HIGHEST)
    o = o + b2_ref[...]
    out_ref[...] = o.reshape(_MBB, _N_MESH, _OUT_CH).astype(jnp.bfloat16)


def _interp_body(idx_ref, wts_ref, mesh_ref, out_ref, a_ref):
    b = pl.program_id(1)

    @pl.when(b == 0)
    def _build_a():
        iota = jax.lax.broadcasted_iota(jnp.int32, (_GB, _N_MESH), 1)
        acc = jnp.zeros((_GB, _N_MESH), jnp.float32)
        for k in range(_N_NEI):
            acc = acc + jnp.where(idx_ref[:, k:k + 1] == iota,
                                  wts_ref[:, k:k + 1], 0.0)
        a_ref[...] = acc.astype(jnp.bfloat16)

    r = jax.lax.dot_general(
        a_ref[...], mesh_ref[0],
        (((1,), (0,)), ((), ())),
        preferred_element_type=jnp.float32)
    out_ref[...] = r.reshape(1, _GLAT, _N_LON, _OUT_CH)


def kernel(node_features, W1, b1, W2, b2, neighbor_indices, neighbor_weights):
    mesh = pl.pallas_call(
        _mlp_body,
        grid=(_BATCH // _MBB,),
        in_specs=[
            pl.BlockSpec((_MBB, _N_MESH, _IN_DIM), lambda i: (i, 0, 0)),
            pl.BlockSpec((_IN_DIM, _HID), lambda i: (0, 0)),
            pl.BlockSpec((1, _HID), lambda i: (0, 0)),
            pl.BlockSpec((_HID, _OUT_CH), lambda i: (0, 0)),
            pl.BlockSpec((1, _OUT_CH), lambda i: (0, 0)),
        ],
        out_specs=pl.BlockSpec((_MBB, _N_MESH, _OUT_CH), lambda i: (i, 0, 0)),
        out_shape=jax.ShapeDtypeStruct((_BATCH, _N_MESH, _OUT_CH),
                                       jnp.bfloat16),
        compiler_params=pltpu.CompilerParams(
            dimension_semantics=("parallel",)),
    )(node_features, W1, b1.reshape(1, _HID), W2, b2.reshape(1, _OUT_CH))

    out = pl.pallas_call(
        _interp_body,
        grid=(_N_LAT // _GLAT, _BATCH),
        in_specs=[
            pl.BlockSpec((_GB, _N_NEI), lambda g, b: (g, 0)),
            pl.BlockSpec((_GB, _N_NEI), lambda g, b: (g, 0)),
            pl.BlockSpec((1, _N_MESH, _OUT_CH), lambda g, b: (b, 0, 0)),
        ],
        out_specs=pl.BlockSpec((1, _GLAT, _N_LON, _OUT_CH),
                               lambda g, b: (b, g, 0, 0)),
        out_shape=jax.ShapeDtypeStruct((_BATCH, _N_LAT, _N_LON, _OUT_CH),
                                       jnp.float32),
        scratch_shapes=[pltpu.VMEM((_GB, _N_MESH), jnp.bfloat16)],
        compiler_params=pltpu.CompilerParams(
            dimension_semantics=("parallel", "arbitrary")),
    )(neighbor_indices, neighbor_weights, mesh)

    return out


# speedup vs baseline: 1.6395x; 1.6395x over previous
import jax
import jax.numpy as jnp
from jax.experimental import pallas as pl
from jax.experimental.pallas import tpu as pltpu

_N_LAT, _N_LON, _N_MESH, _N_NEI = 64, 128, 128, 4
_IN_DIM, _HID, _OUT_CH = 256, 256, 78
_BATCH = 32
_N_GRID = _N_LAT * _N_LON
_MBB = 8
_ABLK = 2048


def _mlp_body(nf_ref, w1_ref, b1_ref, w2_ref, b2_ref, out_ref):
    x = nf_ref[...].reshape(_MBB * _N_MESH, _IN_DIM)
    h = jnp.dot(x, w1_ref[...],
                preferred_element_type=jnp.float32,
                precision=jax.lax.Precision.HIGHEST)
    h = jnp.maximum(h + b1_ref[...], 0.0)
    o = jnp.dot(h, w2_ref[...],
                preferred_element_type=jnp.float32,
                precision=jax.lax.Precision.HIGHEST)
    o = o + b2_ref[...]
    out_ref[...] = o.reshape(_MBB, _N_MESH, _OUT_CH).astype(jnp.bfloat16)


def _build_a_body(idx_ref, wts_ref, a_ref):
    iota = jax.lax.broadcasted_iota(jnp.int32, (_ABLK, _N_MESH), 1)
    acc = jnp.zeros((_ABLK, _N_MESH), jnp.float32)
    for k in range(_N_NEI):
        acc = acc + jnp.where(idx_ref[:, k:k + 1] == iota,
                              wts_ref[:, k:k + 1], 0.0)
    a_ref[...] = acc.astype(jnp.bfloat16)


def _interp_body(a_ref, mesh_ref, out_ref):
    r = jax.lax.dot_general(
        a_ref[...], mesh_ref[0],
        (((1,), (0,)), ((), ())),
        preferred_element_type=jnp.float32)
    out_ref[...] = r.reshape(1, _N_LAT, _N_LON, _OUT_CH)


def kernel(node_features, W1, b1, W2, b2, neighbor_indices, neighbor_weights):
    mesh = pl.pallas_call(
        _mlp_body,
        grid=(_BATCH // _MBB,),
        in_specs=[
            pl.BlockSpec((_MBB, _N_MESH, _IN_DIM), lambda i: (i, 0, 0)),
            pl.BlockSpec((_IN_DIM, _HID), lambda i: (0, 0)),
            pl.BlockSpec((1, _HID), lambda i: (0, 0)),
            pl.BlockSpec((_HID, _OUT_CH), lambda i: (0, 0)),
            pl.BlockSpec((1, _OUT_CH), lambda i: (0, 0)),
        ],
        out_specs=pl.BlockSpec((_MBB, _N_MESH, _OUT_CH), lambda i: (i, 0, 0)),
        out_shape=jax.ShapeDtypeStruct((_BATCH, _N_MESH, _OUT_CH),
                                       jnp.bfloat16),
        compiler_params=pltpu.CompilerParams(
            dimension_semantics=("parallel",)),
    )(node_features, W1, b1.reshape(1, _HID), W2, b2.reshape(1, _OUT_CH))

    a_mat = pl.pallas_call(
        _build_a_body,
        grid=(_N_GRID // _ABLK,),
        in_specs=[
            pl.BlockSpec((_ABLK, _N_NEI), lambda i: (i, 0)),
            pl.BlockSpec((_ABLK, _N_NEI), lambda i: (i, 0)),
        ],
        out_specs=pl.BlockSpec((_ABLK, _N_MESH), lambda i: (i, 0)),
        out_shape=jax.ShapeDtypeStruct((_N_GRID, _N_MESH), jnp.bfloat16),
        compiler_params=pltpu.CompilerParams(
            dimension_semantics=("parallel",)),
    )(neighbor_indices, neighbor_weights)

    out = pl.pallas_call(
        _interp_body,
        grid=(_BATCH,),
        in_specs=[
            pl.BlockSpec((_N_GRID, _N_MESH), lambda b: (0, 0)),
            pl.BlockSpec((1, _N_MESH, _OUT_CH), lambda b: (b, 0, 0)),
        ],
        out_specs=pl.BlockSpec((1, _N_LAT, _N_LON, _OUT_CH),
                               lambda b: (b, 0, 0, 0)),
        out_shape=jax.ShapeDtypeStruct((_BATCH, _N_LAT, _N_LON, _OUT_CH),
                                       jnp.float32),
        compiler_params=pltpu.CompilerParams(
            dimension_semantics=("arbitrary",)),
    )(a_mat, mesh)

    return out


# speedup vs baseline: 1.7749x; 1.0825x over previous
import jax
import jax.numpy as jnp
from jax.experimental import pallas as pl
from jax.experimental.pallas import tpu as pltpu

_N_LAT, _N_LON, _N_MESH, _N_NEI = 64, 128, 128, 4
_IN_DIM, _HID, _OUT_CH = 256, 256, 78
_BATCH = 32
_N_GRID = _N_LAT * _N_LON


def _fused_body(nf_ref, w1_ref, b1_ref, w2_ref, b2_ref, idx_ref, wts_ref,
                out_ref, mesh_s, a_s):
    s = pl.program_id(0)

    @pl.when(s == 0)
    def _prep():
        for c in range(4):
            x = nf_ref[c * 8:(c + 1) * 8].reshape(8 * _N_MESH, _IN_DIM)
            h = jnp.dot(x, w1_ref[...],
                        preferred_element_type=jnp.float32,
                        precision=jax.lax.Precision.HIGHEST)
            h = jnp.maximum(h + b1_ref[...], 0.0)
            o = jnp.dot(h, w2_ref[...],
                        preferred_element_type=jnp.float32,
                        precision=jax.lax.Precision.HIGHEST)
            o = o + b2_ref[...]
            mesh_s[c * 8:(c + 1) * 8] = (
                o.reshape(8, _N_MESH, _OUT_CH).astype(jnp.bfloat16))

        iota = jax.lax.broadcasted_iota(jnp.int32, (_N_GRID // 4, _N_MESH), 1)
        for c in range(4):
            rows = pl.ds(c * (_N_GRID // 4), _N_GRID // 4)
            acc = jnp.zeros((_N_GRID // 4, _N_MESH), jnp.float32)
            for k in range(_N_NEI):
                acc = acc + jnp.where(idx_ref[rows, k:k + 1] == iota,
                                      wts_ref[rows, k:k + 1], 0.0)
            a_s[rows] = acc.astype(jnp.bfloat16)

    @pl.when(s > 0)
    def _interp():
        b = s - 1
        r = jax.lax.dot_general(
            a_s[...], mesh_s[b],
            (((1,), (0,)), ((), ())),
            preferred_element_type=jnp.float32)
        out_ref[...] = r.reshape(1, _N_LAT, _N_LON, _OUT_CH)


def kernel(node_features, W1, b1, W2, b2, neighbor_indices, neighbor_weights):
    out = pl.pallas_call(
        _fused_body,
        grid=(_BATCH + 1,),
        in_specs=[
            pl.BlockSpec((_BATCH, _N_MESH, _IN_DIM), lambda s: (0, 0, 0)),
            pl.BlockSpec((_IN_DIM, _HID), lambda s: (0, 0)),
            pl.BlockSpec((1, _HID), lambda s: (0, 0)),
            pl.BlockSpec((_HID, _OUT_CH), lambda s: (0, 0)),
            pl.BlockSpec((1, _OUT_CH), lambda s: (0, 0)),
            pl.BlockSpec((_N_GRID, _N_NEI), lambda s: (0, 0)),
            pl.BlockSpec((_N_GRID, _N_NEI), lambda s: (0, 0)),
        ],
        out_specs=pl.BlockSpec((1, _N_LAT, _N_LON, _OUT_CH),
                               lambda s: (jnp.maximum(s - 1, 0), 0, 0, 0)),
        out_shape=jax.ShapeDtypeStruct((_BATCH, _N_LAT, _N_LON, _OUT_CH),
                                       jnp.float32),
        scratch_shapes=[
            pltpu.VMEM((_BATCH, _N_MESH, _OUT_CH), jnp.bfloat16),
            pltpu.VMEM((_N_GRID, _N_MESH), jnp.bfloat16),
        ],
        compiler_params=pltpu.CompilerParams(
            dimension_semantics=("arbitrary",)),
    )(node_features, W1, b1.reshape(1, _HID), W2, b2.reshape(1, _OUT_CH),
      neighbor_indices, neighbor_weights)

    return out
